# Initial kernel scaffold; baseline (speedup 1.0000x reference)
import jax
import jax.numpy as jnp
from jax.experimental import pallas as pl


def kernel(x, edge_index, edge_attr, Wl1, Wr1, We1, att1, b1, Ws, bs, gamma, beta, Wl2, Wr2, We2, att2, b2, Wc1, bc1, Wt1, bt1, Wc2, bc2, Wt2, bt2):
    raise NotImplementedError("write your pallas kernel here")


# speedup vs baseline: 18.7900x; 18.7900x over previous
import functools
import math

import jax
import jax.numpy as jnp
from jax import lax
from jax.experimental import pallas as pl
from jax.experimental.pallas import tpu as pltpu
from jax.experimental.pallas import tpu_sc as plsc

_DH = 16
_NC = 2
_NS = 16
_NW = _NC * _NS
_B = 128
_BNC = 1.0 / math.sqrt(1.0 + 1e-5)


def _ceil_div(a, b):
    return -(-a // b)


@functools.lru_cache(maxsize=None)
def _make_edge_pass(n, e, emit_ex):
    assert e % _B == 0
    nblocks = e // _B
    iters = _ceil_div(nblocks, _NW)

    numch = 125
    assert n % numch == 0
    nchunks = n // numch
    num_iters = _ceil_div(nchunks, _NS)
    dench = 2000
    assert n % dench == 0
    dchunks = n // dench
    den_iters = _ceil_div(dchunks, _NS)

    mesh = plsc.VectorSubcoreMesh(core_axis_name="c", subcore_axis_name="s")
    out_type = [
        jax.ShapeDtypeStruct((_NC, n, _DH), jnp.float32),
        jax.ShapeDtypeStruct((_NC, n), jnp.float32),
    ]
    if emit_ex:
        out_type.append(jax.ShapeDtypeStruct((e,), jnp.float32))
    scratch_types = [
        pltpu.VMEM((_B,), jnp.int32),
        pltpu.VMEM((_B,), jnp.int32),
        pltpu.VMEM((_B,), jnp.float32),
        pltpu.VMEM((_B, _DH), jnp.float32),
        pltpu.VMEM((_B, _DH), jnp.float32),
        pltpu.VMEM((_B, _DH), jnp.float32),
        pltpu.VMEM((_B,), jnp.float32),
        pltpu.VMEM((_DH, _DH), jnp.float32),
        pltpu.VMEM((_DH, _DH), jnp.float32),
        pltpu.VMEM((numch, _DH), jnp.float32),
        pltpu.VMEM((dench,), jnp.float32),
        pltpu.VMEM_SHARED((n, _DH), jnp.float32),
        pltpu.VMEM_SHARED((n,), jnp.float32),
        pltpu.SemaphoreType.DMA,
        pltpu.SemaphoreType.DMA,
    ]

    def body(*refs):
        if emit_ex:
            (src_h, dst_h, ea_h, xl_h, xr_h, att_h, we_h,
             num_o, den_o, ex_o,
             srcv, dstv, eav, xlr, xrr, numb, exb, attb, webb,
             zrows, zden, num_sh, den_sh, sem1, sem2) = refs
        else:
            (src_h, dst_h, ea_h, xl_h, xr_h, att_h, we_h,
             num_o, den_o,
             srcv, dstv, eav, xlr, xrr, numb, exb, attb, webb,
             zrows, zden, num_sh, den_sh, sem1, sem2) = refs
            ex_o = None

        c = lax.axis_index("c")
        tid = lax.axis_index("s")
        wid = tid * _NC + c

        pltpu.sync_copy(att_h, attb)
        pltpu.sync_copy(we_h, webb)

        zero16 = jnp.zeros((16,), jnp.float32)
        for r in range(numch):
            zrows[r, :] = zero16
        for i in range(dench // 16):
            zden[pl.ds(i * 16, 16)] = zero16

        def zero_num(i, _):
            cid = i * _NS + tid

            @pl.when(cid < nchunks)
            def _():
                pltpu.sync_copy(zrows, num_sh.at[pl.ds(cid * numch, numch)])
            return 0

        lax.fori_loop(0, num_iters, zero_num, 0)

        def zero_den(i, _):
            cid = i * _NS + tid

            @pl.when(cid < dchunks)
            def _():
                pltpu.sync_copy(zden, den_sh.at[pl.ds(cid * dench, dench)])
            return 0

        lax.fori_loop(0, den_iters, zero_den, 0)

        plsc.subcore_barrier()

        iota16 = lax.iota(jnp.int32, 16)

        def edge_block(i, _):
            blk = i * _NW + wid

            @pl.when(blk < nblocks)
            def _():
                base = blk * _B
                pltpu.sync_copy(src_h.at[pl.ds(base, _B)], srcv)
                pltpu.sync_copy(dst_h.at[pl.ds(base, _B)], dstv)
                pltpu.sync_copy(ea_h.at[pl.ds(base, _B)], eav)
                cp1 = pltpu.async_copy(xl_h.at[srcv], xlr, sem1)
                cp2 = pltpu.async_copy(xr_h.at[dstv], xrr, sem2)
                cp1.wait()
                cp2.wait()
                for sb in range(_B // 16):
                    rows = iota16 + (sb * 16)
                    ea16 = eav[pl.ds(sb * 16, 16)]
                    acc = jnp.zeros((16,), jnp.float32)
                    cols = []
                    for d in range(_DH):
                        dsp = jnp.full((16,), d, jnp.int32)
                        cl = plsc.load_gather(xlr, [rows, dsp])
                        cr = plsc.load_gather(xrr, [rows, dsp])
                        wd = webb[d, :]
                        ad = attb[d, :]
                        sv = cl + cr + ea16 * wd
                        m = jnp.maximum(sv, 0.2 * sv)
                        acc = acc + m * ad
                        cols.append(cl)
                    exv = jnp.exp(acc)
                    exb[pl.ds(sb * 16, 16)] = exv
                    for d in range(_DH):
                        dsp = jnp.full((16,), d, jnp.int32)
                        plsc.store_scatter(numb, [rows, dsp], cols[d] * exv)
                pltpu.sync_copy(exb, den_sh.at[dstv], add=True)
                pltpu.sync_copy(numb, num_sh.at[dstv], add=True)
                if emit_ex:
                    pltpu.sync_copy(exb, ex_o.at[pl.ds(base, _B)])
            return 0

        lax.fori_loop(0, iters, edge_block, 0)

        plsc.subcore_barrier()

        def copy_num(i, _):
            cid = i * _NS + tid

            @pl.when(cid < nchunks)
            def _():
                r0 = cid * numch
                pltpu.sync_copy(num_sh.at[pl.ds(r0, numch)],
                                num_o.at[c, pl.ds(r0, numch)])
            return 0

        lax.fori_loop(0, num_iters, copy_num, 0)

        def copy_den(i, _):
            cid = i * _NS + tid

            @pl.when(cid < dchunks)
            def _():
                d0 = cid * dench
                pltpu.sync_copy(den_sh.at[pl.ds(d0, dench)],
                                den_o.at[c, pl.ds(d0, dench)])
            return 0

        lax.fori_loop(0, den_iters, copy_den, 0)

    return pl.kernel(
        body, out_type=out_type, mesh=mesh, scratch_types=scratch_types,
        compiler_params=pltpu.CompilerParams(use_tc_tiling_on_sc=False, needs_layout_passes=False))


@functools.lru_cache(maxsize=None)
def _make_alpha_pass(n, e):
    assert e % _B == 0
    nblocks = e // _B
    iters = _ceil_div(nblocks, _NW)
    mesh = plsc.VectorSubcoreMesh(core_axis_name="c", subcore_axis_name="s")

    scratch_types = [
        pltpu.VMEM((_B,), jnp.int32),
        pltpu.VMEM((_B,), jnp.float32),
        pltpu.VMEM((_B,), jnp.float32),
        pltpu.VMEM((_B,), jnp.float32),
        pltpu.SemaphoreType.DMA,
    ]

    def body(dst_h, ex_h, den_h, al_o, dstv, exv, denr, alb, sem):
        c = lax.axis_index("c")
        tid = lax.axis_index("s")
        wid = tid * _NC + c

        def edge_block(i, _):
            blk = i * _NW + wid

            @pl.when(blk < nblocks)
            def _():
                base = blk * _B
                pltpu.sync_copy(dst_h.at[pl.ds(base, _B)], dstv)
                pltpu.sync_copy(ex_h.at[pl.ds(base, _B)], exv)
                pltpu.async_copy(den_h.at[dstv], denr, sem).wait()
                for sb in range(_B // 16):
                    ex16 = exv[pl.ds(sb * 16, 16)]
                    dn16 = denr[pl.ds(sb * 16, 16)]
                    alb[pl.ds(sb * 16, 16)] = ex16 / (dn16 + 1e-16)
                pltpu.sync_copy(alb, al_o.at[pl.ds(base, _B)])
            return 0

        lax.fori_loop(0, iters, edge_block, 0)

    return pl.kernel(
        body, out_type=jax.ShapeDtypeStruct((e,), jnp.float32),
        mesh=mesh, scratch_types=scratch_types,
        compiler_params=pltpu.CompilerParams(use_tc_tiling_on_sc=False, needs_layout_passes=False))


def _node_lin_body(x_r, wl_r, wr_r, ws_r, bs_r, xl_o, xr_o, xs_o):
    x = x_r[...]
    xl_o[...] = jnp.dot(x, wl_r[...], preferred_element_type=jnp.float32)
    xr_o[...] = jnp.dot(x, wr_r[...], preferred_element_type=jnp.float32)
    xs_o[...] = (jnp.dot(x, ws_r[...], preferred_element_type=jnp.float32)
                 + bs_r[...])


def _node_lin(x, wlt, wrt, wst, bs2):
    n = x.shape[0]
    g = 100
    bn = n // g
    spec16 = pl.BlockSpec((bn, _DH), lambda i: (i, 0))
    return pl.pallas_call(
        _node_lin_body,
        grid=(g,),
        in_specs=[
            pl.BlockSpec((bn, 2), lambda i: (i, 0)),
            pl.BlockSpec((2, _DH), lambda i: (0, 0)),
            pl.BlockSpec((2, _DH), lambda i: (0, 0)),
            pl.BlockSpec((2, _DH), lambda i: (0, 0)),
            pl.BlockSpec((1, _DH), lambda i: (0, 0)),
        ],
        out_specs=[spec16, spec16, spec16],
        out_shape=[jax.ShapeDtypeStruct((n, _DH), jnp.float32)] * 3,
    )(x, wlt, wrt, wst, bs2)


def _ea_sum_body(ea_r, out_r):
    @pl.when(pl.program_id(0) == 0)
    def _():
        out_r[...] = jnp.zeros_like(out_r)

    out_r[...] += jnp.sum(ea_r[...])


def _ea_sum(ea):
    e = ea.shape[0]
    assert e % 128 == 0
    ea2 = ea.reshape(e // 128, 128)
    g = 25
    assert (e // 128) % g == 0
    be = e // 128 // g
    return pl.pallas_call(
        _ea_sum_body,
        grid=(g,),
        in_specs=[pl.BlockSpec((be, 128), lambda i: (i, 0))],
        out_specs=pl.BlockSpec((1, 1), lambda i: (0, 0)),
        out_shape=jax.ShapeDtypeStruct((1, 1), jnp.float32),
    )(ea2)


def _combine(nump, denp, xl, xr, easum, we, att, inv_e):
    ea_mean = easum[0, 0] * inv_e
    s = xl + xr + ea_mean * we
    m = jnp.maximum(s, 0.2 * s)
    logit = jnp.sum(m * att, axis=1, keepdims=True)
    exl = jnp.exp(logit)
    num = nump[0] + nump[1] + exl * xl
    den = (denp[:, 0] + denp[:, 1])[:, None] + exl
    return num, den, exl


def _make_mid_body(inv_e):
    def body(nump_r, denp_r, xl_r, xr_r, xs_r, easum_r, we_r, att_r, b1_r,
             gamma_r, beta_r, wl2t_r, wr2t_r, xl2_o, xr2_o):
        num, den, _ = _combine(nump_r[...], denp_r[...], xl_r[...], xr_r[...],
                               easum_r[...], we_r[...], att_r[...], inv_e)
        out1 = num / (den + 1e-16) + b1_r[...] + xs_r[...]
        h = gamma_r[...] * (out1 * _BNC) + beta_r[...]
        h = jnp.where(h > 0, h, jnp.exp(h) - 1.0)
        xl2_o[...] = jnp.dot(h, wl2t_r[...],
                             preferred_element_type=jnp.float32)
        xr2_o[...] = jnp.dot(h, wr2t_r[...],
                             preferred_element_type=jnp.float32)
    return body


def _mid(nump, denp, xl1, xr1, xs, easum, we, att, b1, gamma, beta,
         wl2t, wr2t, inv_e):
    n = xl1.shape[0]
    g = 100
    bn = n // g
    spec16 = pl.BlockSpec((bn, _DH), lambda i: (i, 0))
    param = pl.BlockSpec((1, _DH), lambda i: (0, 0))
    w16 = pl.BlockSpec((_DH, _DH), lambda i: (0, 0))
    return pl.pallas_call(
        _make_mid_body(inv_e),
        grid=(g,),
        in_specs=[
            pl.BlockSpec((_NC, bn, _DH), lambda i: (0, i, 0)),
            pl.BlockSpec((bn, _NC), lambda i: (i, 0)),
            spec16, spec16, spec16,
            pl.BlockSpec((1, 1), lambda i: (0, 0)),
            param, param, param, param, param,
            w16, w16,
        ],
        out_specs=[spec16, spec16],
        out_shape=[jax.ShapeDtypeStruct((n, _DH), jnp.float32)] * 2,
    )(nump, denp, xl1, xr1, xs, easum, we, att, b1, gamma, beta, wl2t, wr2t)


def _make_head_body(inv_e):
    def body(nump_r, denp_r, xl_r, xr_r, easum_r, we_r, att_r, b2_r,
             wc1t_r, bc1_r, wc2t_r, bc2_r, wt1t_r, bt1_r, wt2t_r, bt2_r,
             y_o, den_o, aloop_o):
        num, den, exl = _combine(nump_r[...], denp_r[...], xl_r[...],
                                 xr_r[...], easum_r[...], we_r[...],
                                 att_r[...], inv_e)
        out2 = num / (den + 1e-16) + b2_r[...]
        h = jnp.where(out2 > 0, out2, jnp.exp(out2) - 1.0)
        hc = jnp.dot(h, wc1t_r[...], preferred_element_type=jnp.float32) \
            + bc1_r[...]
        clone = jnp.dot(hc, wc2t_r[...], preferred_element_type=jnp.float32) \
            + bc2_r[...]
        ht = jnp.dot(h, wt1t_r[...], preferred_element_type=jnp.float32) \
            + bt1_r[...]
        tpart = jnp.dot(ht, wt2t_r[...], preferred_element_type=jnp.float32) \
            + bt2_r[...]
        y_o[...] = jnp.concatenate([clone, tpart], axis=1)
        den_o[...] = den
        aloop_o[...] = exl / (den + 1e-16)
    return body


def _head(nump, denp, xl2, xr2, easum, we, att, b2,
          wc1t, bc1, wc2t, bc2, wt1t, bt1, wt2t, bt2, inv_e):
    n = xl2.shape[0]
    g = 100
    bn = n // g
    spec16 = pl.BlockSpec((bn, _DH), lambda i: (i, 0))
    param = pl.BlockSpec((1, _DH), lambda i: (0, 0))
    w16 = pl.BlockSpec((_DH, _DH), lambda i: (0, 0))
    return pl.pallas_call(
        _make_head_body(inv_e),
        grid=(g,),
        in_specs=[
            pl.BlockSpec((_NC, bn, _DH), lambda i: (0, i, 0)),
            pl.BlockSpec((bn, _NC), lambda i: (i, 0)),
            spec16, spec16,
            pl.BlockSpec((1, 1), lambda i: (0, 0)),
            param, param, param,
            w16, param,
            pl.BlockSpec((_DH, 9), lambda i: (0, 0)),
            pl.BlockSpec((1, 9), lambda i: (0, 0)),
            w16, param,
            pl.BlockSpec((_DH, 4), lambda i: (0, 0)),
            pl.BlockSpec((1, 4), lambda i: (0, 0)),
        ],
        out_specs=[
            pl.BlockSpec((bn, 13), lambda i: (i, 0)),
            pl.BlockSpec((bn, 1), lambda i: (i, 0)),
            pl.BlockSpec((bn, 1), lambda i: (i, 0)),
        ],
        out_shape=[
            jax.ShapeDtypeStruct((n, 13), jnp.float32),
            jax.ShapeDtypeStruct((n, 1), jnp.float32),
            jax.ShapeDtypeStruct((n, 1), jnp.float32),
        ],
    )(nump, denp, xl2, xr2, easum, we, att, b2,
      wc1t, bc1, wc2t, bc2, wt1t, bt1, wt2t, bt2)


def kernel(x, edge_index, edge_attr, Wl1, Wr1, We1, att1, b1, Ws, bs,
           gamma, beta, Wl2, Wr2, We2, att2, b2, Wc1, bc1, Wt1, bt1,
           Wc2, bc2, Wt2, bt2):
    n = x.shape[0]
    e = edge_index.shape[1]
    inv_e = 1.0 / e

    src = edge_index[0]
    dst = edge_index[1]
    ea = edge_attr[:, 0]

    xl1, xr1, xs = _node_lin(x, Wl1.T, Wr1.T, Ws.T, bs[None])
    easum = _ea_sum(edge_attr)

    we1 = We1[:, 0]
    att1b = jnp.broadcast_to(att1[:, None], (_DH, _DH))
    we1b = jnp.broadcast_to(we1[:, None], (_DH, _DH))
    nump1, denp1 = _make_edge_pass(n, e, False)(
        src, dst, ea, xl1, xr1, att1b, we1b)
    denp1 = denp1.T

    xl2, xr2 = _mid(nump1, denp1, xl1, xr1, xs, easum, we1[None], att1[None],
                    b1[None], gamma[None], beta[None], Wl2.T, Wr2.T, inv_e)

    we2 = We2[:, 0]
    att2b = jnp.broadcast_to(att2[:, None], (_DH, _DH))
    we2b = jnp.broadcast_to(we2[:, None], (_DH, _DH))
    nump2, denp2, ex2 = _make_edge_pass(n, e, True)(
        src, dst, ea, xl2, xr2, att2b, we2b)
    denp2 = denp2.T

    y, den2, aloop = _head(nump2, denp2, xl2, xr2, easum, we2[None],
                           att2[None], b2[None], Wc1.T, bc1[None], Wc2.T,
                           bc2[None], Wt1.T, bt1[None], Wt2.T, bt2[None],
                           inv_e)

    alpha_e = _make_alpha_pass(n, e)(dst, ex2, den2.reshape(-1))
    alpha = jnp.concatenate([alpha_e, aloop.reshape(-1)])
    return (y, alpha)
